# Initial kernel scaffold; baseline (speedup 1.0000x reference)
import jax
import jax.numpy as jnp
from jax.experimental import pallas as pl


def kernel(node_type_ids, edge_index, edge_types, hyperedge_members, hyperedge_types, hyperedge_mask, params):
    raise NotImplementedError("write your pallas kernel here")


# speedup vs baseline: 6.2302x; 6.2302x over previous
import functools
import math

import jax
import jax.numpy as jnp
from jax.experimental import pallas as pl
from jax.experimental.pallas import tpu as pltpu

_D = 256
_H = 8
_HD = 32
_N = 1024
_NE = 1024
_AR = 8
_NT = 32
_ET = 16


def _gelu(x):
  return 0.5 * x * (1.0 + jax.lax.erf(x * (1.0 / math.sqrt(2.0))))


def _ln(x, g, b):
  m = jnp.mean(x, axis=-1, keepdims=True)
  v = jnp.mean((x - m) ** 2, axis=-1, keepdims=True)
  return (x - m) * jax.lax.rsqrt(v + 1e-5) * g + b


def _embed_body(ids_ref, emb_ref, o_ref):
  ids = ids_ref[0]
  tid = jax.lax.broadcasted_iota(jnp.int32, (_NT, _N), 0)
  oh = jnp.where(ids == tid, 1.0, 0.0)
  o_ref[0] = jax.lax.dot_general(
      oh, emb_ref[...], (((0,), (0,)), ((), ())),
      preferred_element_type=jnp.float32)


def _embed(ids, emb, B):
  return pl.pallas_call(
      _embed_body,
      grid=(B,),
      in_specs=[
          pl.BlockSpec((1, 1, _N), lambda b: (b, 0, 0)),
          pl.BlockSpec((_NT, _D), lambda b: (0, 0)),
      ],
      out_specs=pl.BlockSpec((1, _N, _D), lambda b: (b, 0, 0)),
      out_shape=jax.ShapeDtypeStruct((B, _N, _D), jnp.float32),
  )(ids.reshape(B, 1, _N), emb)


def _qkv_body(x_ref, wp_ref, bp_ref, wq_ref, bq_ref, wk_ref, bk_ref,
              wv_ref, bv_ref, q_ref, k_ref, v_ref):
  x = x_ref[...]
  h = jnp.dot(x, wp_ref[...], preferred_element_type=jnp.float32) + bp_ref[...]
  q_ref[...] = jnp.dot(h, wq_ref[...], preferred_element_type=jnp.float32) + bq_ref[...]
  k_ref[...] = jnp.dot(h, wk_ref[...], preferred_element_type=jnp.float32) + bk_ref[...]
  v_ref[...] = jnp.dot(h, wv_ref[...], preferred_element_type=jnp.float32) + bv_ref[...]


def _qkv(x2d, p, rows, bm):
  nblk = rows // bm
  w_spec = pl.BlockSpec((_D, _D), lambda i: (0, 0))
  b_spec = pl.BlockSpec((1, _D), lambda i: (0, 0))
  r_spec = pl.BlockSpec((bm, _D), lambda i: (i, 0))
  return pl.pallas_call(
      _qkv_body,
      grid=(nblk,),
      in_specs=[r_spec, w_spec, b_spec, w_spec, b_spec, w_spec, b_spec,
                w_spec, b_spec],
      out_specs=[r_spec, r_spec, r_spec],
      out_shape=[jax.ShapeDtypeStruct((rows, _D), jnp.float32)] * 3,
  )(x2d, p['node_proj_w'], p['node_proj_b'].reshape(1, _D),
    p['query_w'], p['query_b'].reshape(1, _D),
    p['key_w'], p['key_b'].reshape(1, _D),
    p['value_w'], p['value_b'].reshape(1, _D))


def _attn_body(q_ref, k_ref, v_ref, o_ref):
  for h in range(_H):
    sl = slice(h * _HD, (h + 1) * _HD)
    q = q_ref[0, :, sl]
    k = k_ref[0, :, sl]
    v = v_ref[0, :, sl]
    s = jax.lax.dot_general(q, k, (((1,), (1,)), ((), ())),
                            preferred_element_type=jnp.float32)
    s = s * (1.0 / math.sqrt(_HD))
    m = jnp.max(s, axis=-1, keepdims=True)
    e = jnp.exp(s - m)
    p = e / jnp.sum(e, axis=-1, keepdims=True)
    o_ref[0, :, sl] = jnp.dot(p, v, preferred_element_type=jnp.float32)


def _attention(q, k, v, B):
  spec = pl.BlockSpec((1, _N, _D), lambda b: (b, 0, 0))
  return pl.pallas_call(
      _attn_body,
      grid=(B,),
      in_specs=[spec, spec, spec],
      out_specs=spec,
      out_shape=jax.ShapeDtypeStruct((B, _N, _D), jnp.float32),
  )(q.reshape(B, _N, _D), k.reshape(B, _N, _D), v.reshape(B, _N, _D))


def _proj_body(a_ref, r_ref, w_ref, b_ref, g_ref, bb_ref, o_ref):
  o = jnp.dot(a_ref[...], w_ref[...], preferred_element_type=jnp.float32)
  o = o + b_ref[...] + r_ref[...]
  o_ref[...] = _ln(o, g_ref[...], bb_ref[...])


def _proj_ln(attn_out, resid, p, rows, bm):
  nblk = rows // bm
  r_spec = pl.BlockSpec((bm, _D), lambda i: (i, 0))
  w_spec = pl.BlockSpec((_D, _D), lambda i: (0, 0))
  b_spec = pl.BlockSpec((1, _D), lambda i: (0, 0))
  return pl.pallas_call(
      _proj_body,
      grid=(nblk,),
      in_specs=[r_spec, r_spec, w_spec, b_spec, b_spec, b_spec],
      out_specs=r_spec,
      out_shape=jax.ShapeDtypeStruct((rows, _D), jnp.float32),
  )(attn_out, resid, p['out_proj_w'], p['out_proj_b'].reshape(1, _D),
    p['norm_g'].reshape(1, _D), p['norm_b'].reshape(1, _D))


def _ffn_body(x_ref, w1_ref, b1_ref, w2_ref, b2_ref, g_ref, bb_ref, o_ref):
  x = x_ref[...]
  h = _gelu(jnp.dot(x, w1_ref[...], preferred_element_type=jnp.float32)
            + b1_ref[...])
  y = jnp.dot(h, w2_ref[...], preferred_element_type=jnp.float32) + b2_ref[...]
  o_ref[...] = _ln(y + x, g_ref[...], bb_ref[...])


def _ffn(x2d, p, rows, bm):
  nblk = rows // bm
  r_spec = pl.BlockSpec((bm, _D), lambda i: (i, 0))
  return pl.pallas_call(
      _ffn_body,
      grid=(nblk,),
      in_specs=[
          r_spec,
          pl.BlockSpec((_D, 4 * _D), lambda i: (0, 0)),
          pl.BlockSpec((1, 4 * _D), lambda i: (0, 0)),
          pl.BlockSpec((4 * _D, _D), lambda i: (0, 0)),
          pl.BlockSpec((1, _D), lambda i: (0, 0)),
          pl.BlockSpec((1, _D), lambda i: (0, 0)),
          pl.BlockSpec((1, _D), lambda i: (0, 0)),
      ],
      out_specs=r_spec,
      out_shape=jax.ShapeDtypeStruct((rows, _D), jnp.float32),
  )(x2d, p['ff1_w'], p['ff1_b'].reshape(1, 4 * _D), p['ff2_w'],
    p['ff2_b'].reshape(1, _D), p['ffn_g'].reshape(1, _D),
    p['ffn_b'].reshape(1, _D))


def _hyp_body(x_ref, mem_ref, maskf_ref, types_ref,
              et_ref, ew1_ref, ew2_ref, eb_ref, eg_ref, ebb_ref,
              uw1_ref, uw2_ref, ub_ref, ug_ref, ubb_ref, o_ref):
  x = x_ref[0]
  mem = mem_ref[0]
  maskf = maskf_ref[0]
  nid = jax.lax.broadcasted_iota(jnp.int32, (_NE, _N), 1)
  w = jnp.zeros((_NE, _N), jnp.float32)
  for a in range(_AR):
    hit = mem[:, a:a + 1] == nid
    w = w + jnp.where(hit, maskf[:, a:a + 1], 0.0)
  cnt = jnp.clip(jnp.sum(maskf, axis=-1, keepdims=True), 1.0)
  pooled = jnp.dot(w, x, preferred_element_type=jnp.float32) / cnt

  types = types_ref[0]
  eid = jax.lax.broadcasted_iota(jnp.int32, (_ET, _NE), 0)
  eoh = jnp.where(types == eid, 1.0, 0.0)
  edge_emb = jax.lax.dot_general(
      eoh, et_ref[...], (((0,), (0,)), ((), ())),
      preferred_element_type=jnp.float32)

  ef = (jnp.dot(pooled, ew1_ref[...], preferred_element_type=jnp.float32)
        + jnp.dot(edge_emb, ew2_ref[...], preferred_element_type=jnp.float32)
        + eb_ref[...])
  ef = _ln(_gelu(ef), eg_ref[...], ebb_ref[...])

  counts = jax.lax.dot_general(
      w, jnp.ones((_NE, 1), jnp.float32), (((0,), (0,)), ((), ())),
      preferred_element_type=jnp.float32)
  counts = jnp.clip(counts, 1.0)
  nup = jax.lax.dot_general(
      w, ef, (((0,), (0,)), ((), ())),
      preferred_element_type=jnp.float32) / counts

  upd = (jnp.dot(x, uw1_ref[...], preferred_element_type=jnp.float32)
         + jnp.dot(nup, uw2_ref[...], preferred_element_type=jnp.float32)
         + ub_ref[...])
  o_ref[0] = _ln(_gelu(upd), ug_ref[...], ubb_ref[...])


def _hyp(x, members, maskf, types, p, B):
  full = lambda *s: pl.BlockSpec(s, lambda b: (0,) * len(s))
  batch3 = lambda d1, d2: pl.BlockSpec((1, d1, d2), lambda b: (b, 0, 0))
  ew1 = p['enc_w'][:_D]
  ew2 = p['enc_w'][_D:]
  uw1 = p['upd_w'][:_D]
  uw2 = p['upd_w'][_D:]
  return pl.pallas_call(
      _hyp_body,
      grid=(B,),
      in_specs=[
          batch3(_N, _D),
          batch3(_NE, _AR),
          batch3(_NE, _AR),
          batch3(1, _NE),
          full(_ET, 32),
          full(_D, _D), full(32, _D), full(1, _D), full(1, _D), full(1, _D),
          full(_D, _D), full(_D, _D), full(1, _D), full(1, _D), full(1, _D),
      ],
      out_specs=batch3(_N, _D),
      out_shape=jax.ShapeDtypeStruct((B, _N, _D), jnp.float32),
  )(x, members, maskf, types.reshape(B, 1, _NE), p['he_etype'],
    ew1, ew2, p['enc_b'].reshape(1, _D), p['enc_g'].reshape(1, _D),
    p['enc_bb'].reshape(1, _D),
    uw1, uw2, p['upd_b'].reshape(1, _D), p['upd_g'].reshape(1, _D),
    p['upd_bb'].reshape(1, _D))


def _pool_body(x_ref, w_ref, b_ref, g_ref, bb_ref, o_ref):
  gm = jnp.mean(x_ref[...], axis=1)
  gm = jnp.dot(gm, w_ref[...], preferred_element_type=jnp.float32) + b_ref[...]
  o_ref[...] = _ln(_gelu(gm), g_ref[...], bb_ref[...])


def _pool(x, params, B):
  return pl.pallas_call(
      _pool_body,
      in_specs=[
          pl.BlockSpec((B, _N, _D), lambda: (0, 0, 0)),
          pl.BlockSpec((_D, _D), lambda: (0, 0)),
          pl.BlockSpec((1, _D), lambda: (0, 0)),
          pl.BlockSpec((1, _D), lambda: (0, 0)),
          pl.BlockSpec((1, _D), lambda: (0, 0)),
      ],
      out_specs=pl.BlockSpec((B, _D), lambda: (0, 0)),
      out_shape=jax.ShapeDtypeStruct((B, _D), jnp.float32),
  )(x, params['pool_w'], params['pool_b'].reshape(1, _D),
    params['pool_g'].reshape(1, _D), params['pool_bb'].reshape(1, _D))


def kernel(node_type_ids, edge_index, edge_types, hyperedge_members,
           hyperedge_types, hyperedge_mask, params):
  del edge_index, edge_types
  B = node_type_ids.shape[0]
  rows = B * _N
  bm = 1024

  ids = node_type_ids.astype(jnp.int32)
  members = hyperedge_members.astype(jnp.int32)
  types = hyperedge_types.astype(jnp.int32)
  maskf = hyperedge_mask.astype(jnp.float32)

  x = _embed(ids, params['node_type_embed'], B)
  for p in params['layers']:
    x2d = x.reshape(rows, _D)
    q, k, v = _qkv(x2d, p, rows, bm)
    attn_out = _attention(q, k, v, B)
    x2d = _proj_ln(attn_out.reshape(rows, _D), x2d, p, rows, bm)
    x2d = _ffn(x2d, p, rows, bm)
    x = _hyp(x2d.reshape(B, _N, _D), members, maskf, types, p, B)
  graph_emb = _pool(x, params, B)
  return x, graph_emb
